# baseline (speedup 1.0000x reference)
import functools

import jax
import jax.numpy as jnp
from jax import lax
from jax.experimental import pallas as pl
from jax.experimental.pallas import tpu as pltpu
from jax.experimental.pallas import tpu_sc as plsc

N = 10000
E = 320000
D = 128
N2 = 10240
R = 5120
GRID = N2 // R
NC, NS = 2, 16
K = 125
CH = 80
CHH = CH // 2
RPT = N2 // NS
WB = 80


def _mesh():
    return plsc.VectorSubcoreMesh(
        core_axis_name="c", subcore_axis_name="s",
        num_cores=NC, num_subcores=NS)


def _sc_degree(dst2):
    @functools.partial(
        pl.kernel,
        out_type=jax.ShapeDtypeStruct((NC, N2), jnp.float32),
        mesh=_mesh(),
        scratch_types=[
            pltpu.VMEM((CH, K), jnp.int32),
            pltpu.VMEM((128,), jnp.float32),
            pltpu.VMEM((RPT,), jnp.float32),
            pltpu.VMEM_SHARED((N2,), jnp.float32),
        ],
    )
    def body(dst_hbm, deg_out, didx, ones_v, buf_v, deg_sp):
        cid = lax.axis_index("c")
        sid = lax.axis_index("s")
        tid = cid * NS + sid
        for i in range(8):
            ones_v[pl.ds(i * 16, 16)] = jnp.full((16,), 1.0, jnp.float32)

        def zb(i, _):
            buf_v[pl.ds(i * 16, 16)] = jnp.zeros((16,), jnp.float32)
            return 0

        lax.fori_loop(0, RPT // 16, zb, 0)
        pltpu.sync_copy(buf_v, deg_sp.at[pl.ds(sid * RPT, RPT)])
        pltpu.sync_copy(dst_hbm.at[pl.ds(tid * CH, CH)], didx)
        plsc.subcore_barrier()

        def step(j, _):
            pltpu.sync_copy(ones_v.at[pl.ds(0, K)], deg_sp.at[didx.at[j]],
                            add=True)
            return 0

        lax.fori_loop(0, CH, step, 0)
        plsc.subcore_barrier()
        pltpu.sync_copy(deg_sp.at[pl.ds(sid * RPT, RPT)],
                        deg_out.at[cid, pl.ds(sid * RPT, RPT)])

    return body(dst2)


def _sc_aggregate(src2, dst2, hs, zblk):
    @functools.partial(
        pl.kernel,
        out_type=jax.ShapeDtypeStruct((NC, N2, D), jnp.float32),
        mesh=_mesh(),
        scratch_types=[
            pltpu.VMEM((CHH, K), jnp.int32),
            pltpu.VMEM((CHH, K), jnp.int32),
            pltpu.VMEM((K, D), jnp.float32),
            pltpu.VMEM((K, D), jnp.float32),
            pltpu.VMEM_SHARED((N2, D), jnp.float32),
            pltpu.SemaphoreType.DMA,
            pltpu.SemaphoreType.DMA,
            pltpu.SemaphoreType.DMA,
            pltpu.SemaphoreType.DMA,
        ],
    )
    def body(src_hbm, dst_hbm, hs_hbm, z_hbm, p_out,
             sidx, didx, r0, r1, acc_sp, gs0, gs1, ss0, ss1):
        cid = lax.axis_index("c")
        sid = lax.axis_index("s")
        tid = cid * NS + sid
        rows = (r0, r1)
        pltpu.sync_copy(src_hbm.at[pl.ds(tid * CH, CHH)], sidx)
        pltpu.async_copy(hs_hbm.at[sidx.at[0]], r0, gs0)
        pltpu.sync_copy(dst_hbm.at[pl.ds(tid * CH, CHH)], didx)
        pltpu.sync_copy(z_hbm, r1.at[pl.ds(0, WB)])
        for k in range(RPT // WB):
            pltpu.sync_copy(r1.at[pl.ds(0, WB)],
                            acc_sp.at[pl.ds(sid * RPT + k * WB, WB)])
        plsc.subcore_barrier()

        for h in range(2):
            if h == 1:
                pltpu.sync_copy(
                    src_hbm.at[pl.ds(tid * CH + CHH, CHH)], sidx)
                pltpu.sync_copy(
                    dst_hbm.at[pl.ds(tid * CH + CHH, CHH)], didx)
                pltpu.async_copy(hs_hbm.at[sidx.at[0]], r0, gs0)

            def step(g, _):
                even = (g % 2) == 0
                nxt = g + 1

                @pl.when((g >= 1) & even)
                def _():
                    pltpu.make_async_copy(
                        r1, acc_sp.at[didx.at[g - 1]], ss1).wait()

                @pl.when((g >= 1) & jnp.logical_not(even))
                def _():
                    pltpu.make_async_copy(
                        r0, acc_sp.at[didx.at[g - 1]], ss0).wait()

                @pl.when((nxt < CHH) & even)
                def _():
                    pltpu.async_copy(hs_hbm.at[sidx.at[nxt]], r1, gs1)

                @pl.when((nxt < CHH) & jnp.logical_not(even))
                def _():
                    pltpu.async_copy(hs_hbm.at[sidx.at[nxt]], r0, gs0)

                @pl.when(even)
                def _():
                    pltpu.make_async_copy(
                        hs_hbm.at[sidx.at[g]], r0, gs0).wait()
                    pltpu.async_copy(r0, acc_sp.at[didx.at[g]], ss0, add=True)

                @pl.when(jnp.logical_not(even))
                def _():
                    pltpu.make_async_copy(
                        hs_hbm.at[sidx.at[g]], r1, gs1).wait()
                    pltpu.async_copy(r1, acc_sp.at[didx.at[g]], ss1, add=True)

                return 0

            lax.fori_loop(0, CHH, step, 0)
            pltpu.make_async_copy(r1, acc_sp.at[didx.at[CHH - 1]], ss1).wait()
        plsc.subcore_barrier()
        pltpu.sync_copy(acc_sp.at[pl.ds(sid * RPT, RPT)],
                        p_out.at[cid, pl.ds(sid * RPT, RPT)])

    return body(src2, dst2, hs, zblk)


def _tc_prescale(x_p, W1, degp3):
    def body(x_ref, w_ref, degp_ref, hs_ref, dinv_ref):
        deg = degp_ref[0] + degp_ref[1] + 1.0
        dinv = lax.rsqrt(deg)
        h = jnp.dot(x_ref[...], w_ref[...], preferred_element_type=jnp.float32)
        hs_ref[...] = h * dinv
        dinv_ref[...] = dinv

    return pl.pallas_call(
        body,
        grid=(GRID,),
        in_specs=[
            pl.BlockSpec((R, D), lambda i: (i, 0)),
            pl.BlockSpec((D, D), lambda i: (0, 0)),
            pl.BlockSpec((NC, R, 1), lambda i: (0, i, 0)),
        ],
        out_specs=[
            pl.BlockSpec((R, D), lambda i: (i, 0)),
            pl.BlockSpec((R, 1), lambda i: (i, 0)),
        ],
        out_shape=[
            jax.ShapeDtypeStruct((N2, D), jnp.float32),
            jax.ShapeDtypeStruct((N2, 1), jnp.float32),
        ],
    )(x_p, W1, degp3)


def _tc_layer_mid(p, hs1, dinv, b1, W2):
    def body(p_ref, hs_ref, dinv_ref, b_ref, w_ref, out_ref):
        agg = p_ref[0] + p_ref[1] + hs_ref[...]
        o1 = jnp.maximum(agg * dinv_ref[...] + b_ref[...], 0.0)
        out_ref[...] = jnp.dot(
            o1, w_ref[...], preferred_element_type=jnp.float32) * dinv_ref[...]

    return pl.pallas_call(
        body,
        grid=(GRID,),
        in_specs=[
            pl.BlockSpec((NC, R, D), lambda i: (0, i, 0)),
            pl.BlockSpec((R, D), lambda i: (i, 0)),
            pl.BlockSpec((R, 1), lambda i: (i, 0)),
            pl.BlockSpec((1, D), lambda i: (0, 0)),
            pl.BlockSpec((D, D), lambda i: (0, 0)),
        ],
        out_specs=pl.BlockSpec((R, D), lambda i: (i, 0)),
        out_shape=jax.ShapeDtypeStruct((N2, D), jnp.float32),
    )(p, hs1, dinv, b1, W2)


def _tc_final(q, hs2, dinv, b2):
    def body(q_ref, hs_ref, dinv_ref, b_ref, out_ref):
        agg = q_ref[0] + q_ref[1] + hs_ref[...]
        out_ref[...] = agg * dinv_ref[...] + b_ref[...]

    return pl.pallas_call(
        body,
        grid=(GRID,),
        in_specs=[
            pl.BlockSpec((NC, R, D), lambda i: (0, i, 0)),
            pl.BlockSpec((R, D), lambda i: (i, 0)),
            pl.BlockSpec((R, 1), lambda i: (i, 0)),
            pl.BlockSpec((1, D), lambda i: (0, 0)),
        ],
        out_specs=pl.BlockSpec((R, D), lambda i: (i, 0)),
        out_shape=jax.ShapeDtypeStruct((N2, D), jnp.float32),
    )(q, hs2, dinv, b2)


def kernel(x, edge_index, W1, b1, W2, b2):
    ei = edge_index.astype(jnp.int32)
    src_p = ei[0].reshape(NC * NS * CH, K)
    dst_p = ei[1].reshape(NC * NS * CH, K)
    x_p = jnp.pad(x, ((0, N2 - N), (0, 0)))
    zblk = jnp.zeros((WB, D), jnp.float32)

    degp = _sc_degree(dst_p)
    degp3 = degp.reshape(NC, N2, 1)
    hs1, dinv = _tc_prescale(x_p, W1, degp3)
    p = _sc_aggregate(src_p, dst_p, hs1, zblk)
    hs2 = _tc_layer_mid(p, hs1, dinv, b1.reshape(1, D), W2)
    q = _sc_aggregate(src_p, dst_p, hs2, zblk)
    out = _tc_final(q, hs2, dinv, b2.reshape(1, D))
    return out[:N]


# speedup vs baseline: 1.0030x; 1.0030x over previous
import functools

import jax
import jax.numpy as jnp
from jax import lax
from jax.experimental import pallas as pl
from jax.experimental.pallas import tpu as pltpu
from jax.experimental.pallas import tpu_sc as plsc

N = 10000
E = 320000
D = 128
N2 = 10240
R = 2560
GRID = N2 // R
NC, NS = 2, 16
K = 125
CH = 80
CHH = CH // 2
RPT = N2 // NS
WB = 80


def _mesh():
    return plsc.VectorSubcoreMesh(
        core_axis_name="c", subcore_axis_name="s",
        num_cores=NC, num_subcores=NS)


def _sc_degree(dst2):
    @functools.partial(
        pl.kernel,
        out_type=jax.ShapeDtypeStruct((NC, N2), jnp.float32),
        mesh=_mesh(),
        scratch_types=[
            pltpu.VMEM((CH, K), jnp.int32),
            pltpu.VMEM((128,), jnp.float32),
            pltpu.VMEM((RPT,), jnp.float32),
            pltpu.VMEM_SHARED((N2,), jnp.float32),
        ],
    )
    def body(dst_hbm, deg_out, didx, ones_v, buf_v, deg_sp):
        cid = lax.axis_index("c")
        sid = lax.axis_index("s")
        tid = cid * NS + sid
        for i in range(8):
            ones_v[pl.ds(i * 16, 16)] = jnp.full((16,), 1.0, jnp.float32)

        def zb(i, _):
            buf_v[pl.ds(i * 16, 16)] = jnp.zeros((16,), jnp.float32)
            return 0

        lax.fori_loop(0, RPT // 16, zb, 0)
        pltpu.sync_copy(buf_v, deg_sp.at[pl.ds(sid * RPT, RPT)])
        pltpu.sync_copy(dst_hbm.at[pl.ds(tid * CH, CH)], didx)
        plsc.subcore_barrier()

        def step(j, _):
            pltpu.sync_copy(ones_v.at[pl.ds(0, K)], deg_sp.at[didx.at[j]],
                            add=True)
            return 0

        lax.fori_loop(0, CH, step, 0)
        plsc.subcore_barrier()
        pltpu.sync_copy(deg_sp.at[pl.ds(sid * RPT, RPT)],
                        deg_out.at[cid, pl.ds(sid * RPT, RPT)])

    return body(dst2)


def _sc_aggregate(src2, dst2, hs, zblk):
    @functools.partial(
        pl.kernel,
        out_type=jax.ShapeDtypeStruct((NC, N2, D), jnp.float32),
        mesh=_mesh(),
        scratch_types=[
            pltpu.VMEM((CHH, K), jnp.int32),
            pltpu.VMEM((CHH, K), jnp.int32),
            pltpu.VMEM((K, D), jnp.float32),
            pltpu.VMEM((K, D), jnp.float32),
            pltpu.VMEM_SHARED((N2, D), jnp.float32),
            pltpu.SemaphoreType.DMA,
            pltpu.SemaphoreType.DMA,
            pltpu.SemaphoreType.DMA,
            pltpu.SemaphoreType.DMA,
        ],
    )
    def body(src_hbm, dst_hbm, hs_hbm, z_hbm, p_out,
             sidx, didx, r0, r1, acc_sp, gs0, gs1, ss0, ss1):
        cid = lax.axis_index("c")
        sid = lax.axis_index("s")
        tid = cid * NS + sid
        rows = (r0, r1)
        pltpu.sync_copy(src_hbm.at[pl.ds(tid * CH, CHH)], sidx)
        pltpu.async_copy(hs_hbm.at[sidx.at[0]], r0, gs0)
        pltpu.sync_copy(dst_hbm.at[pl.ds(tid * CH, CHH)], didx)
        pltpu.sync_copy(z_hbm, r1.at[pl.ds(0, WB)])
        for k in range(RPT // WB):
            pltpu.sync_copy(r1.at[pl.ds(0, WB)],
                            acc_sp.at[pl.ds(sid * RPT + k * WB, WB)])
        plsc.subcore_barrier()

        for h in range(2):
            if h == 1:
                pltpu.sync_copy(
                    src_hbm.at[pl.ds(tid * CH + CHH, CHH)], sidx)
                pltpu.sync_copy(
                    dst_hbm.at[pl.ds(tid * CH + CHH, CHH)], didx)
                pltpu.async_copy(hs_hbm.at[sidx.at[0]], r0, gs0)

            def step(g, _):
                even = (g % 2) == 0
                nxt = g + 1

                @pl.when((g >= 1) & even)
                def _():
                    pltpu.make_async_copy(
                        r1, acc_sp.at[didx.at[g - 1]], ss1).wait()

                @pl.when((g >= 1) & jnp.logical_not(even))
                def _():
                    pltpu.make_async_copy(
                        r0, acc_sp.at[didx.at[g - 1]], ss0).wait()

                @pl.when((nxt < CHH) & even)
                def _():
                    pltpu.async_copy(hs_hbm.at[sidx.at[nxt]], r1, gs1)

                @pl.when((nxt < CHH) & jnp.logical_not(even))
                def _():
                    pltpu.async_copy(hs_hbm.at[sidx.at[nxt]], r0, gs0)

                @pl.when(even)
                def _():
                    pltpu.make_async_copy(
                        hs_hbm.at[sidx.at[g]], r0, gs0).wait()
                    pltpu.async_copy(r0, acc_sp.at[didx.at[g]], ss0, add=True)

                @pl.when(jnp.logical_not(even))
                def _():
                    pltpu.make_async_copy(
                        hs_hbm.at[sidx.at[g]], r1, gs1).wait()
                    pltpu.async_copy(r1, acc_sp.at[didx.at[g]], ss1, add=True)

                return 0

            lax.fori_loop(0, CHH, step, 0)
            pltpu.make_async_copy(r1, acc_sp.at[didx.at[CHH - 1]], ss1).wait()
        plsc.subcore_barrier()
        pltpu.sync_copy(acc_sp.at[pl.ds(sid * RPT, RPT)],
                        p_out.at[cid, pl.ds(sid * RPT, RPT)])

    return body(src2, dst2, hs, zblk)


def _tc_prescale(x_p, W1, degp3):
    def body(x_ref, w_ref, degp_ref, hs_ref, dinv_ref):
        deg = degp_ref[0] + degp_ref[1] + 1.0
        dinv = lax.rsqrt(deg)
        h = jnp.dot(x_ref[...], w_ref[...], preferred_element_type=jnp.float32)
        hs_ref[...] = h * dinv
        dinv_ref[...] = dinv

    return pl.pallas_call(
        body,
        grid=(GRID,),
        in_specs=[
            pl.BlockSpec((R, D), lambda i: (i, 0)),
            pl.BlockSpec((D, D), lambda i: (0, 0)),
            pl.BlockSpec((NC, R, 1), lambda i: (0, i, 0)),
        ],
        out_specs=[
            pl.BlockSpec((R, D), lambda i: (i, 0)),
            pl.BlockSpec((R, 1), lambda i: (i, 0)),
        ],
        out_shape=[
            jax.ShapeDtypeStruct((N2, D), jnp.float32),
            jax.ShapeDtypeStruct((N2, 1), jnp.float32),
        ],
    )(x_p, W1, degp3)


def _tc_layer_mid(p, hs1, dinv, b1, W2):
    def body(p_ref, hs_ref, dinv_ref, b_ref, w_ref, out_ref):
        agg = p_ref[0] + p_ref[1] + hs_ref[...]
        o1 = jnp.maximum(agg * dinv_ref[...] + b_ref[...], 0.0)
        out_ref[...] = jnp.dot(
            o1, w_ref[...], preferred_element_type=jnp.float32) * dinv_ref[...]

    return pl.pallas_call(
        body,
        grid=(GRID,),
        in_specs=[
            pl.BlockSpec((NC, R, D), lambda i: (0, i, 0)),
            pl.BlockSpec((R, D), lambda i: (i, 0)),
            pl.BlockSpec((R, 1), lambda i: (i, 0)),
            pl.BlockSpec((1, D), lambda i: (0, 0)),
            pl.BlockSpec((D, D), lambda i: (0, 0)),
        ],
        out_specs=pl.BlockSpec((R, D), lambda i: (i, 0)),
        out_shape=jax.ShapeDtypeStruct((N2, D), jnp.float32),
    )(p, hs1, dinv, b1, W2)


def _tc_final(q, hs2, dinv, b2):
    def body(q_ref, hs_ref, dinv_ref, b_ref, out_ref):
        agg = q_ref[0] + q_ref[1] + hs_ref[...]
        out_ref[...] = agg * dinv_ref[...] + b_ref[...]

    return pl.pallas_call(
        body,
        grid=(GRID,),
        in_specs=[
            pl.BlockSpec((NC, R, D), lambda i: (0, i, 0)),
            pl.BlockSpec((R, D), lambda i: (i, 0)),
            pl.BlockSpec((R, 1), lambda i: (i, 0)),
            pl.BlockSpec((1, D), lambda i: (0, 0)),
        ],
        out_specs=pl.BlockSpec((R, D), lambda i: (i, 0)),
        out_shape=jax.ShapeDtypeStruct((N2, D), jnp.float32),
    )(q, hs2, dinv, b2)


def kernel(x, edge_index, W1, b1, W2, b2):
    ei = edge_index.astype(jnp.int32)
    src_p = ei[0].reshape(NC * NS * CH, K)
    dst_p = ei[1].reshape(NC * NS * CH, K)
    x_p = jnp.pad(x, ((0, N2 - N), (0, 0)))
    zblk = jnp.zeros((WB, D), jnp.float32)

    degp = _sc_degree(dst_p)
    degp3 = degp.reshape(NC, N2, 1)
    hs1, dinv = _tc_prescale(x_p, W1, degp3)
    p = _sc_aggregate(src_p, dst_p, hs1, zblk)
    hs2 = _tc_layer_mid(p, hs1, dinv, b1.reshape(1, D), W2)
    q = _sc_aggregate(src_p, dst_p, hs2, zblk)
    out = _tc_final(q, hs2, dinv, b2.reshape(1, D))
    return out[:N]
